# Initial kernel scaffold; baseline (speedup 1.0000x reference)
import jax
import jax.numpy as jnp
from jax.experimental import pallas as pl


def kernel(queries, keys):
    raise NotImplementedError("write your pallas kernel here")


# speedup vs baseline: 5.3923x; 5.3923x over previous
import jax
import jax.numpy as jnp
from jax.experimental import pallas as pl
from jax.experimental.pallas import tpu as pltpu

_N = 8192
_BQ = 256
_BIG = 1e10
_R2_DST = 34.0 * 34.0
_R2_SRC = 10.0 * 10.0
_K_DST = 64
_K_SRC = 8


def _select_topk(d2_ref, iota, K, dd_ref, di_ref):
    kio = jax.lax.broadcasted_iota(jnp.int32, (_BQ, K), 1)

    def body(k, _):
        d2 = d2_ref[...]
        m = jnp.min(d2, axis=1, keepdims=True)
        eq = d2 == m
        cand = jnp.where(eq, iota, _N)
        am = jnp.min(cand, axis=1, keepdims=True)
        valid = m < _BIG
        dval = jnp.where(valid, m, 0.0)
        ival = jnp.where(valid, am, -1)
        sel = kio == k
        dd_ref[...] = jnp.where(sel, dval, dd_ref[...])
        di_ref[...] = jnp.where(sel, ival, di_ref[...])
        d2_ref[...] = jnp.where(iota == am, _BIG, d2)
        return 0

    jax.lax.fori_loop(0, K, body, 0)


def _tc_body(q_ref, kT_ref, qT_ref, dd_ref, di_ref, sd_ref, si_ref, d2_ref):
    q = q_ref[...]
    iota = jax.lax.broadcasted_iota(jnp.int32, (_BQ, _N), 1)
    for kT, r2, K, ddr, dir_ in (
        (kT_ref, _R2_DST, _K_DST, dd_ref, di_ref),
        (qT_ref, _R2_SRC, _K_SRC, sd_ref, si_ref),
    ):
        kt = kT[...]
        qk = jax.lax.dot_general(
            q, kt, (((1,), (0,)), ((), ())), preferred_element_type=jnp.float32
        )
        q2 = jnp.sum(q * q, axis=1, keepdims=True)
        k2 = jnp.sum(kt * kt, axis=0, keepdims=True)
        d2 = jnp.maximum(q2 + k2 - 2.0 * qk, 0.0)
        d2_ref[...] = jnp.where(d2 <= r2, d2, _BIG)
        _select_topk(d2_ref, iota, K, ddr, dir_)


def kernel(queries, keys):
    kT = keys.T
    qT = queries.T
    grid = _N // _BQ
    out_shapes = [
        jax.ShapeDtypeStruct((_N, _K_DST), jnp.float32),
        jax.ShapeDtypeStruct((_N, _K_DST), jnp.int32),
        jax.ShapeDtypeStruct((_N, _K_SRC), jnp.float32),
        jax.ShapeDtypeStruct((_N, _K_SRC), jnp.int32),
    ]
    dd, di, sd, si = pl.pallas_call(
        _tc_body,
        grid=(grid,),
        in_specs=[
            pl.BlockSpec((_BQ, 2), lambda i: (i, 0)),
            pl.BlockSpec((2, _N), lambda i: (0, 0)),
            pl.BlockSpec((2, _N), lambda i: (0, 0)),
        ],
        out_specs=[
            pl.BlockSpec((_BQ, _K_DST), lambda i: (i, 0)),
            pl.BlockSpec((_BQ, _K_DST), lambda i: (i, 0)),
            pl.BlockSpec((_BQ, _K_SRC), lambda i: (i, 0)),
            pl.BlockSpec((_BQ, _K_SRC), lambda i: (i, 0)),
        ],
        out_shape=out_shapes,
        scratch_shapes=[pltpu.VMEM((_BQ, _N), jnp.float32)],
    )(queries, kT, qT)
    return dd, di, sd, si


# speedup vs baseline: 6.2240x; 1.1542x over previous
import functools

import jax
import jax.numpy as jnp
from jax import lax
from jax.experimental import pallas as pl
from jax.experimental.pallas import tpu as pltpu
from jax.experimental.pallas import tpu_sc as plsc

_N = 8192
_BIG = 1e10
_R2_DST = 34.0 * 34.0
_R2_SRC = 10.0 * 10.0
_NW = 32
_QPW = _N // _NW


def _lex_less(ak, av, bk, bv):
    return (ak < bk) | ((ak == bk) & (av < bv))


def _ce(ak, av, bk, bv):
    c = _lex_less(ak, av, bk, bv)
    lok = jnp.where(c, ak, bk)
    lov = jnp.where(c, av, bv)
    hik = jnp.where(c, bk, ak)
    hiv = jnp.where(c, bv, av)
    return lok, lov, hik, hiv


def _rev(k, v):
    return lax.rev(k, (0,)), lax.rev(v, (0,))


_SH = []


def _butterfly(k, v, j, take_min):
    shk_ref, shv_ref = _SH
    perm = lax.iota(jnp.int32, 16) ^ j
    shk_ref[...] = k
    shv_ref[...] = v
    pk = plsc.load_gather(shk_ref, [perm])
    pv = plsc.load_gather(shv_ref, [perm])
    c = _lex_less(k, v, pk, pv)
    choose_self = jnp.where(take_min, c, ~c)
    return jnp.where(choose_self, k, pk), jnp.where(choose_self, v, pv)


def _sort16(k, v):
    lane = lax.iota(jnp.int32, 16)
    for size in (2, 4, 8, 16):
        up = (lane & size) == 0
        j = size // 2
        while j >= 1:
            lower = (lane & j) == 0
            k, v = _butterfly(k, v, j, lower == up)
            j //= 2
    return k, v


def _bmerge16(k, v):
    lane = lax.iota(jnp.int32, 16)
    for j in (8, 4, 2, 1):
        k, v = _butterfly(k, v, j, (lane & j) == 0)
    return k, v


def _merge2(ak, av, bk, bv):
    rbk, rbv = _rev(bk, bv)
    lok, lov, hik, hiv = _ce(ak, av, rbk, rbv)
    lok, lov = _bmerge16(lok, lov)
    hik, hiv = _bmerge16(hik, hiv)
    return lok, lov, hik, hiv


def _bm2(k0, v0, k1, v1):
    lok, lov, hik, hiv = _ce(k0, v0, k1, v1)
    lok, lov = _bmerge16(lok, lov)
    hik, hiv = _bmerge16(hik, hiv)
    return lok, lov, hik, hiv


def _bm4(k0, v0, k1, v1, k2, v2, k3, v3):
    l0k, l0v, h0k, h0v = _ce(k0, v0, k2, v2)
    l1k, l1v, h1k, h1v = _ce(k1, v1, k3, v3)
    a = _bm2(l0k, l0v, l1k, l1v)
    b = _bm2(h0k, h0v, h1k, h1v)
    return a + b


def _sort64(k0, v0, k1, v1, k2, v2, k3, v3):
    k0, v0 = _sort16(k0, v0)
    k1, v1 = _sort16(k1, v1)
    k2, v2 = _sort16(k2, v2)
    k3, v3 = _sort16(k3, v3)
    a0k, a0v, a1k, a1v = _merge2(k0, v0, k1, v1)
    b0k, b0v, b1k, b1v = _merge2(k2, v2, k3, v3)
    rb0k, rb0v = _rev(b1k, b1v)
    rb1k, rb1v = _rev(b0k, b0v)
    l0k, l0v, h0k, h0v = _ce(a0k, a0v, rb0k, rb0v)
    l1k, l1v, h1k, h1v = _ce(a1k, a1v, rb1k, rb1v)
    lo = _bm2(l0k, l0v, l1k, l1v)
    hi = _bm2(h0k, h0v, h1k, h1v)
    return lo + hi


def _merge_low64(bufk, bufv, ck, cv):
    b = []
    for t in range(4):
        rk, rv = _rev(ck[3 - t], cv[3 - t])
        lok, lov, _, _ = _ce(bufk[t], bufv[t], rk, rv)
        b.append((lok, lov))
    s = _bm4(b[0][0], b[0][1], b[1][0], b[1][1],
             b[2][0], b[2][1], b[3][0], b[3][1])
    return (s[0], s[2], s[4], s[6]), (s[1], s[3], s[5], s[7])


def _scan_candidates(qxb, qyb, q2, xb_ref, yb_ref, n2_ref, idx_ref,
                     cd_ref, ci_ref, r2):
    def chunk(c, nc):
        o = c * 16
        kx = xb_ref[pl.ds(o, 16)]
        ky = yb_ref[pl.ds(o, 16)]
        k2 = n2_ref[pl.ds(o, 16)]
        kqi = idx_ref[pl.ds(o, 16)]
        qk = qxb * kx + qyb * ky
        d2 = jnp.maximum((q2 + k2) - 2.0 * qk, 0.0)
        m = d2 <= r2
        mi = m.astype(jnp.int32)
        offs = nc + jnp.cumsum(mi) - mi
        plsc.store_scatter(cd_ref, [offs], d2, mask=m)
        plsc.store_scatter(ci_ref, [offs], kqi, mask=m)
        return nc + jnp.sum(mi)

    return lax.fori_loop(0, _N // 16, chunk, jnp.int32(0))


def _pad64(cd_ref, ci_ref, nc):
    iot = lax.iota(jnp.int32, 16)
    big = jnp.full((16,), _BIG, jnp.float32)
    neg = jnp.full((16,), -1, jnp.int32)
    for j in range(4):
        offs = nc + j * 16 + iot
        plsc.store_scatter(cd_ref, [offs], big)
        plsc.store_scatter(ci_ref, [offs], neg)


def _topk_dst(cd_ref, ci_ref, nc, od_ref, oi_ref, obase):
    big = jnp.full((16,), _BIG, jnp.float32)
    neg = jnp.full((16,), -1, jnp.int32)
    init = (big, big, big, big, neg, neg, neg, neg)
    nchunks = (nc + 63) >> 6

    def body(j, buf):
        o = j * 64
        c = [(cd_ref[pl.ds(o + 16 * t, 16)], ci_ref[pl.ds(o + 16 * t, 16)])
             for t in range(4)]
        s = _sort64(c[0][0], c[0][1], c[1][0], c[1][1],
                    c[2][0], c[2][1], c[3][0], c[3][1])
        ck = (s[0], s[2], s[4], s[6])
        cv = (s[1], s[3], s[5], s[7])
        nk, nv = _merge_low64(buf[:4], buf[4:], ck, cv)
        return nk + nv

    buf = lax.fori_loop(0, nchunks, body, init)
    for t in range(4):
        bk, bv = buf[t], buf[4 + t]
        valid = bk < _BIG
        od_ref[pl.ds(obase + 16 * t, 16)] = jnp.where(valid, bk, 0.0)
        oi_ref[pl.ds(obase + 16 * t, 16)] = jnp.where(valid, bv, -1)


def _topk_src(cd_ref, ci_ref, nc, od_ref, oi_ref, obase):
    big = jnp.full((16,), _BIG, jnp.float32)
    neg = jnp.full((16,), -1, jnp.int32)
    nchunks = (nc + 15) >> 4

    def body(j, buf):
        bk, bv = buf
        ck = cd_ref[pl.ds(j * 16, 16)]
        cv = ci_ref[pl.ds(j * 16, 16)]
        ck, cv = _sort16(ck, cv)
        rk, rv = _rev(ck, cv)
        lok, lov, _, _ = _ce(bk, bv, rk, rv)
        return _bmerge16(lok, lov)

    bk, bv = lax.fori_loop(0, nchunks, body, (big, neg))
    valid = bk < _BIG
    od_ref[pl.ds(obase, 16)] = jnp.where(valid, bk, 0.0)
    oi_ref[pl.ds(obase, 16)] = jnp.where(valid, bv, -1)


def _sc_body(qxb_h, qyb_h, q2_h, kxb_h, kyb_h, k2_h, idx_h,
             odd_h, odi_h, osd_h, osi_h,
             qxb_v, qyb_v, q2_v, kxb_v, kyb_v, k2_v, idx_v,
             cd_v, ci_v, dd_v, di_v, sd_v, si_v, shk_v, shv_v):
    _SH[:] = [shk_v, shv_v]
    wid = lax.axis_index("s") * 2 + lax.axis_index("c")
    pltpu.sync_copy(qxb_h, qxb_v)
    pltpu.sync_copy(qyb_h, qyb_v)
    pltpu.sync_copy(q2_h, q2_v)
    pltpu.sync_copy(kxb_h, kxb_v)
    pltpu.sync_copy(kyb_h, kyb_v)
    pltpu.sync_copy(k2_h, k2_v)
    pltpu.sync_copy(idx_h, idx_v)

    def rnd(ref):
        def b(c, _):
            v = plsc.bitcast(ref[pl.ds(c * 16, 16)], jnp.int32)
            v = (v + 0x7FFF + ((v >> 16) & 1)) & -65536
            ref[pl.ds(c * 16, 16)] = plsc.bitcast(v, jnp.float32)
            return 0
        lax.fori_loop(0, _N // 16, b, 0)

    rnd(qxb_v)
    rnd(qyb_v)
    rnd(kxb_v)
    rnd(kyb_v)
    base = wid * _QPW

    def per_query(r, _):
        i = base + r
        iv = jnp.zeros((16,), jnp.int32) + i
        qxb = plsc.load_gather(qxb_v, [iv])
        qyb = plsc.load_gather(qyb_v, [iv])
        q2 = plsc.load_gather(q2_v, [iv])
        nc = _scan_candidates(qxb, qyb, q2, kxb_v, kyb_v, k2_v, idx_v,
                              cd_v, ci_v, _R2_DST)
        _pad64(cd_v, ci_v, nc)
        _topk_dst(cd_v, ci_v, nc, dd_v, di_v, r * 64)
        ns = _scan_candidates(qxb, qyb, q2, qxb_v, qyb_v, q2_v, idx_v,
                              cd_v, ci_v, _R2_SRC)
        _pad64(cd_v, ci_v, ns)
        _topk_src(cd_v, ci_v, ns, sd_v, si_v, r * 16)
        return 0

    lax.fori_loop(0, _QPW, per_query, 0)
    pltpu.sync_copy(dd_v, odd_h.at[pl.ds(base * 64, _QPW * 64)])
    pltpu.sync_copy(di_v, odi_h.at[pl.ds(base * 64, _QPW * 64)])
    pltpu.sync_copy(sd_v, osd_h.at[pl.ds(base * 16, _QPW * 16)])
    pltpu.sync_copy(si_v, osi_h.at[pl.ds(base * 16, _QPW * 16)])


def kernel(queries, keys):
    f32 = jnp.float32
    qx, qy = queries[:, 0], queries[:, 1]
    kx, ky = keys[:, 0], keys[:, 1]
    q2 = qx * qx + qy * qy
    k2 = kx * kx + ky * ky
    idx = jnp.arange(_N, dtype=jnp.int32)

    mesh = plsc.VectorSubcoreMesh(core_axis_name="c", subcore_axis_name="s")
    run = functools.partial(
        pl.kernel,
        mesh=mesh,
        compiler_params=pltpu.CompilerParams(
            use_tc_tiling_on_sc=False, needs_layout_passes=False
        ),
        out_type=[
            jax.ShapeDtypeStruct((_N * 64,), f32),
            jax.ShapeDtypeStruct((_N * 64,), jnp.int32),
            jax.ShapeDtypeStruct((_N * 16,), f32),
            jax.ShapeDtypeStruct((_N * 16,), jnp.int32),
        ],
        scratch_types=[
            pltpu.VMEM((_N,), f32),
            pltpu.VMEM((_N,), f32),
            pltpu.VMEM((_N,), f32),
            pltpu.VMEM((_N,), f32),
            pltpu.VMEM((_N,), f32),
            pltpu.VMEM((_N,), f32),
            pltpu.VMEM((_N,), jnp.int32),
            pltpu.VMEM((_N + 64,), f32),
            pltpu.VMEM((_N + 64,), jnp.int32),
            pltpu.VMEM((_QPW * 64,), f32),
            pltpu.VMEM((_QPW * 64,), jnp.int32),
            pltpu.VMEM((_QPW * 16,), f32),
            pltpu.VMEM((_QPW * 16,), jnp.int32),
            pltpu.VMEM((16,), f32),
            pltpu.VMEM((16,), jnp.int32),
        ],
    )(_sc_body)
    odd, odi, osd, osi = run(qx, qy, q2, kx, ky, k2, idx)
    dd = odd.reshape(_N, 64)
    di = odi.reshape(_N, 64)
    sd = osd.reshape(_N, 16)[:, :8]
    si = osi.reshape(_N, 16)[:, :8]
    return dd, di, sd, si
